# baseline (speedup 1.0000x reference)
import functools

import jax
import jax.numpy as jnp
from jax import lax
from jax.experimental import pallas as pl
from jax.experimental.pallas import tpu as pltpu
from jax.experimental.pallas import tpu_sc as plsc

BATCH = 16384
NUM_WORKERS = 32
BPW = BATCH // NUM_WORKERS
CHUNK = 128
NCHUNK = BPW // CHUNK
D_MEM, D_PARTY, D_STATE = 32, 16, 16
D_OUT = D_MEM + D_PARTY + D_STATE

MEMBER_ROWS, STATE_ROWS, PARTY_ROWS = 1000000, 100000, 1000
LANE_TILE = 128
NT_M = -(-MEMBER_ROWS // LANE_TILE)
NT_S = -(-STATE_ROWS // LANE_TILE)
NT_P = -(-PARTY_ROWS // LANE_TILE)
PAD_M = NT_M * LANE_TILE
PAD_S = NT_S * LANE_TILE
PAD_P = NT_P * LANE_TILE
DPW_M = -(-NT_M // NUM_WORKERS)
DPW_S = -(-NT_S // NUM_WORKERS)


CT_M = 45
NCH_M = -(-DPW_M // CT_M)


def _detile_body(mtab4, stab2, ptab2, mflat, sflat, pflat,
                 vbuf0, vbuf1, sbuf, rsem0, rsem1, wsem0, wsem1):
    wid = lax.axis_index("s") * 2 + lax.axis_index("c")
    md0 = jnp.minimum(wid * DPW_M, NT_M - DPW_M) * LANE_TILE
    sd0 = jnp.minimum(wid * DPW_S, NT_S - DPW_S) * LANE_TILE
    pd0 = jnp.minimum(wid, NT_P - 1) * LANE_TILE

    CW = CT_M * LANE_TILE
    chunks = []
    for a in range(4):
        for ci in range(NCH_M):
            off = min(ci * CT_M, DPW_M - CT_M) * LANE_TILE
            chunks.append((a, md0 + off))
    bufs = (vbuf0, vbuf1)
    rsems = (rsem0, rsem1)
    wsems = (wsem0, wsem1)
    rcps = [None, None]
    wcps = [[], []]
    for i, (a, start) in enumerate(chunks[:2]):
        rcps[i] = pltpu.async_copy(
            mtab4.at[a, :, pl.ds(start, CW)], bufs[i], rsems[i])
    for i, (a, start) in enumerate(chunks):
        b = i % 2
        rcps[b].wait()
        wcps[b] = [
            pltpu.async_copy(
                bufs[b].at[c],
                mflat.at[pl.ds((a * 8 + c) * PAD_M + start, CW)],
                wsems[b])
            for c in range(8)
        ]
        if i + 2 < len(chunks):
            for w in wcps[b]:
                w.wait()
            wcps[b] = []
            na, nstart = chunks[i + 2]
            rcps[b] = pltpu.async_copy(
                mtab4.at[na, :, pl.ds(nstart, CW)], bufs[b], rsems[b])
    for lst in wcps:
        for w in lst:
            w.wait()

    SW = DPW_S * LANE_TILE
    for a in range(2):
        pltpu.sync_copy(stab2.at[a, :, pl.ds(sd0, SW)], sbuf)
        for c in range(8):
            pltpu.sync_copy(
                sbuf.at[c], sflat.at[pl.ds((a * 8 + c) * PAD_S + sd0, SW)])

    for a in range(2):
        for c in range(8):
            f = a * 8 + c
            pltpu.sync_copy(
                ptab2.at[a, c, pl.ds(pd0, LANE_TILE)],
                pflat.at[pl.ds(f * PAD_P + pd0, LANE_TILE)])


def _gather_body(member_hbm, state_hbm, party_hbm,
                 mtab_hbm, stab_hbm, ptab_hbm, out_hbm,
                 midx_v, sidx_v, pidx_v, outbuf, ptab_v,
                 msem, ssem):
    wid = lax.axis_index("s") * 2 + lax.axis_index("c")
    base = wid * BPW
    row0 = wid * NCHUNK

    pltpu.sync_copy(member_hbm.at[pl.ds(row0, NCHUNK)], midx_v)
    pltpu.sync_copy(state_hbm.at[pl.ds(row0, NCHUNK)], sidx_v)
    pltpu.sync_copy(party_hbm.at[pl.ds(row0, NCHUNK)], pidx_v)
    pltpu.sync_copy(ptab_hbm, ptab_v)

    copies = []
    for j in range(NCHUNK):
        cols = pl.ds(j * CHUNK, CHUNK)
        for f in range(D_MEM):
            copies.append(pltpu.async_copy(
                mtab_hbm.at[f].at[midx_v.at[j]], outbuf.at[f, cols], msem))
        for f in range(D_STATE):
            copies.append(pltpu.async_copy(
                stab_hbm.at[f].at[sidx_v.at[j]],
                outbuf.at[D_MEM + D_PARTY + f, cols], ssem))

    def party_grp(it, _):
        f = it // (BPW // 16)
        g = it % (BPW // 16)
        pv = pidx_v[g // 8, pl.ds((g % 8) * 16, 16)]
        vals = plsc.load_gather(ptab_v, [jnp.full((16,), f, jnp.int32), pv])
        outbuf[D_MEM + f, pl.ds(g * 16, 16)] = vals
        return _

    lax.fori_loop(0, D_PARTY * (BPW // 16), party_grp, 0)

    for c in copies:
        c.wait()

    pltpu.sync_copy(outbuf, out_hbm.at[:, pl.ds(base, BPW)])


@jax.jit
def _mem_encoder_sc(member, state, party, member_table, state_table, party_table):
    mesh = plsc.VectorSubcoreMesh(core_axis_name="c", subcore_axis_name="s")
    detile = functools.partial(
        pl.kernel,
        out_type=(
            jax.ShapeDtypeStruct((D_MEM * PAD_M,), jnp.float32),
            jax.ShapeDtypeStruct((D_STATE * PAD_S,), jnp.float32),
            jax.ShapeDtypeStruct((D_PARTY * PAD_P,), jnp.float32),
        ),
        mesh=mesh,
        scratch_types=[
            pltpu.VMEM((8, CT_M * LANE_TILE), jnp.float32),
            pltpu.VMEM((8, CT_M * LANE_TILE), jnp.float32),
            pltpu.VMEM((8, DPW_S * LANE_TILE), jnp.float32),
            pltpu.SemaphoreType.DMA,
            pltpu.SemaphoreType.DMA,
            pltpu.SemaphoreType.DMA,
            pltpu.SemaphoreType.DMA,
        ],
    )(_detile_body)
    gather = functools.partial(
        pl.kernel,
        out_type=jax.ShapeDtypeStruct((D_OUT, BATCH), jnp.float32),
        mesh=mesh,
        scratch_types=[
            pltpu.VMEM((NCHUNK, CHUNK), jnp.int32),
            pltpu.VMEM((NCHUNK, CHUNK), jnp.int32),
            pltpu.VMEM((NCHUNK, CHUNK), jnp.int32),
            pltpu.VMEM((D_OUT, BPW), jnp.float32),
            pltpu.VMEM((D_PARTY, PAD_P), jnp.float32),
            pltpu.SemaphoreType.DMA,
            pltpu.SemaphoreType.DMA,
        ],
        compiler_params=pltpu.CompilerParams(
            use_tc_tiling_on_sc=False, needs_layout_passes=False),
    )(_gather_body)

    mflat, sflat, pflat = detile(
        member_table.T.reshape(4, 8, MEMBER_ROWS),
        state_table.T.reshape(2, 8, STATE_ROWS),
        party_table.T.reshape(2, 8, PARTY_ROWS),
    )
    member2d = member.astype(jnp.int32).reshape(BATCH // CHUNK, CHUNK)
    state2d = state.astype(jnp.int32).reshape(BATCH // CHUNK, CHUNK)
    party2d = party.astype(jnp.int32).reshape(BATCH // CHUNK, CHUNK)
    out_t = gather(member2d, state2d, party2d,
                   mflat.reshape(D_MEM, PAD_M),
                   sflat.reshape(D_STATE, PAD_S),
                   pflat.reshape(D_PARTY, PAD_P))
    return out_t.T


def kernel(member, state, party, member_table, state_table, party_table):
    return _mem_encoder_sc(member, state, party,
                           member_table, state_table, party_table)


# speedup vs baseline: 1.0311x; 1.0311x over previous
import functools

import jax
import jax.numpy as jnp
from jax import lax
from jax.experimental import pallas as pl
from jax.experimental.pallas import tpu as pltpu
from jax.experimental.pallas import tpu_sc as plsc

BATCH = 16384
NUM_WORKERS = 32
BPW = BATCH // NUM_WORKERS
CHUNK = 128
NCHUNK = BPW // CHUNK
D_MEM, D_PARTY, D_STATE = 32, 16, 16
D_OUT = D_MEM + D_PARTY + D_STATE

MEMBER_ROWS, STATE_ROWS, PARTY_ROWS = 1000000, 100000, 1000
LANE_TILE = 128
NT_M = -(-MEMBER_ROWS // LANE_TILE)
NT_S = -(-STATE_ROWS // LANE_TILE)
NT_P = -(-PARTY_ROWS // LANE_TILE)
PAD_M = NT_M * LANE_TILE
PAD_S = NT_S * LANE_TILE
PAD_P = NT_P * LANE_TILE
DPW_M = -(-NT_M // NUM_WORKERS)
DPW_S = -(-NT_S // NUM_WORKERS)


CT_M = 45
NCH_M = -(-DPW_M // CT_M)


def _detile_body(mtab4, stab2, ptab2, mflat, sflat, pflat,
                 vbuf0, vbuf1, sbuf, rsem0, rsem1, wsem0, wsem1):
    wid = lax.axis_index("s") * 2 + lax.axis_index("c")
    md0 = jnp.minimum(wid * DPW_M, NT_M - DPW_M) * LANE_TILE
    sd0 = jnp.minimum(wid * DPW_S, NT_S - DPW_S) * LANE_TILE
    pd0 = jnp.minimum(wid, NT_P - 1) * LANE_TILE

    CW = CT_M * LANE_TILE
    chunks = []
    for a in range(4):
        for ci in range(NCH_M):
            off = min(ci * CT_M, DPW_M - CT_M) * LANE_TILE
            chunks.append((a, md0 + off))
    bufs = (vbuf0, vbuf1)
    rsems = (rsem0, rsem1)
    wsems = (wsem0, wsem1)
    rcps = [None, None]
    wcps = [[], []]
    for i, (a, start) in enumerate(chunks[:2]):
        rcps[i] = pltpu.async_copy(
            mtab4.at[a, :, pl.ds(start, CW)], bufs[i], rsems[i])
    for i, (a, start) in enumerate(chunks):
        b = i % 2
        rcps[b].wait()
        wcps[b] = [
            pltpu.async_copy(
                bufs[b].at[c],
                mflat.at[pl.ds((a * 8 + c) * PAD_M + start, CW)],
                wsems[b])
            for c in range(8)
        ]
        if i + 2 < len(chunks):
            for w in wcps[b]:
                w.wait()
            wcps[b] = []
            na, nstart = chunks[i + 2]
            rcps[b] = pltpu.async_copy(
                mtab4.at[na, :, pl.ds(nstart, CW)], bufs[b], rsems[b])
    for lst in wcps:
        for w in lst:
            w.wait()

    SW = DPW_S * LANE_TILE
    for a in range(2):
        pltpu.sync_copy(stab2.at[a, :, pl.ds(sd0, SW)], sbuf)
        for c in range(8):
            pltpu.sync_copy(
                sbuf.at[c], sflat.at[pl.ds((a * 8 + c) * PAD_S + sd0, SW)])

    for a in range(2):
        for c in range(8):
            f = a * 8 + c
            pltpu.sync_copy(
                ptab2.at[a, c, pl.ds(pd0, LANE_TILE)],
                pflat.at[pl.ds(f * PAD_P + pd0, LANE_TILE)])


def _gather_body(member_hbm, state_hbm, party_hbm,
                 mtab_hbm, stab_hbm, ptab_hbm, out_hbm,
                 midx_v, sidx_v, pidx_v, outbuf, ptab_v,
                 msem, ssem):
    wid = lax.axis_index("s") * 2 + lax.axis_index("c")
    row0 = wid * NCHUNK

    pltpu.sync_copy(member_hbm.at[pl.ds(row0, NCHUNK)], midx_v)
    pltpu.sync_copy(state_hbm.at[pl.ds(row0, NCHUNK)], sidx_v)
    pltpu.sync_copy(party_hbm.at[pl.ds(row0, NCHUNK)], pidx_v)
    pltpu.sync_copy(ptab_hbm, ptab_v)

    copies = []
    for j in range(NCHUNK):
        for f in range(D_MEM):
            copies.append(pltpu.async_copy(
                mtab_hbm.at[f].at[midx_v.at[j]],
                outbuf.at[f // 8, j, f % 8], msem))
        for f in range(D_STATE):
            fo = D_MEM + D_PARTY + f
            copies.append(pltpu.async_copy(
                stab_hbm.at[f].at[sidx_v.at[j]],
                outbuf.at[fo // 8, j, fo % 8], ssem))

    def party_grp(it, _):
        f = it // (BPW // 16)
        g = it % (BPW // 16)
        fo = D_MEM + f
        pv = pidx_v[g // 8, pl.ds((g % 8) * 16, 16)]
        vals = plsc.load_gather(ptab_v, [jnp.full((16,), f, jnp.int32), pv])
        outbuf[fo // 8, g // 8, fo % 8, pl.ds((g % 8) * 16, 16)] = vals
        return _

    lax.fori_loop(0, D_PARTY * (BPW // 16), party_grp, 0)

    for c in copies:
        c.wait()

    pltpu.sync_copy(outbuf, out_hbm.at[:, pl.ds(wid * NCHUNK, NCHUNK)])


@jax.jit
def _mem_encoder_sc(member, state, party, member_table, state_table, party_table):
    mesh = plsc.VectorSubcoreMesh(core_axis_name="c", subcore_axis_name="s")
    detile = functools.partial(
        pl.kernel,
        out_type=(
            jax.ShapeDtypeStruct((D_MEM * PAD_M,), jnp.float32),
            jax.ShapeDtypeStruct((D_STATE * PAD_S,), jnp.float32),
            jax.ShapeDtypeStruct((D_PARTY * PAD_P,), jnp.float32),
        ),
        mesh=mesh,
        scratch_types=[
            pltpu.VMEM((8, CT_M * LANE_TILE), jnp.float32),
            pltpu.VMEM((8, CT_M * LANE_TILE), jnp.float32),
            pltpu.VMEM((8, DPW_S * LANE_TILE), jnp.float32),
            pltpu.SemaphoreType.DMA,
            pltpu.SemaphoreType.DMA,
            pltpu.SemaphoreType.DMA,
            pltpu.SemaphoreType.DMA,
        ],
    )(_detile_body)
    gather = functools.partial(
        pl.kernel,
        out_type=jax.ShapeDtypeStruct(
            (D_OUT // 8, BATCH // CHUNK, 8, CHUNK), jnp.float32),
        mesh=mesh,
        scratch_types=[
            pltpu.VMEM((NCHUNK, CHUNK), jnp.int32),
            pltpu.VMEM((NCHUNK, CHUNK), jnp.int32),
            pltpu.VMEM((NCHUNK, CHUNK), jnp.int32),
            pltpu.VMEM((D_OUT // 8, NCHUNK, 8, CHUNK), jnp.float32),
            pltpu.VMEM((D_PARTY, PAD_P), jnp.float32),
            pltpu.SemaphoreType.DMA,
            pltpu.SemaphoreType.DMA,
        ],
        compiler_params=pltpu.CompilerParams(
            use_tc_tiling_on_sc=False, needs_layout_passes=False),
    )(_gather_body)

    mflat, sflat, pflat = detile(
        member_table.T.reshape(4, 8, MEMBER_ROWS),
        state_table.T.reshape(2, 8, STATE_ROWS),
        party_table.T.reshape(2, 8, PARTY_ROWS),
    )
    member2d = member.astype(jnp.int32).reshape(BATCH // CHUNK, CHUNK)
    state2d = state.astype(jnp.int32).reshape(BATCH // CHUNK, CHUNK)
    party2d = party.astype(jnp.int32).reshape(BATCH // CHUNK, CHUNK)
    out4 = gather(member2d, state2d, party2d,
                  mflat.reshape(D_MEM, PAD_M),
                  sflat.reshape(D_STATE, PAD_S),
                  pflat.reshape(D_PARTY, PAD_P))
    return out4.transpose(1, 3, 0, 2).reshape(BATCH, D_OUT)


def kernel(member, state, party, member_table, state_table, party_table):
    return _mem_encoder_sc(member, state, party,
                           member_table, state_table, party_table)
